# baseline (speedup 1.0000x reference)
import functools

import jax
import jax.numpy as jnp
from jax import lax
from jax.experimental import pallas as pl
from jax.experimental.pallas import tpu as pltpu
from jax.experimental.pallas import tpu_sc as plsc

SEQ = 8192
BATCH = 4
D = 1024
NW = 32
ROWS_PER_W = SEQ // NW
CHUNK = 8
NCHUNK = ROWS_PER_W // CHUNK
DCH = D // 16


def _sc_body(x_hbm, pos_hbm, out_hbm, x_buf, pos_buf):
    wid = lax.axis_index("s") * 2 + lax.axis_index("c")
    row0 = wid * ROWS_PER_W

    def chunk_body(ci, _):
        base = row0 + ci * CHUNK
        pltpu.sync_copy(x_hbm.at[pl.ds(base, CHUNK)], x_buf)
        pltpu.sync_copy(pos_hbm.at[pl.ds(base, CHUNK)], pos_buf)

        def add_body(i, _):
            r = i // DCH
            off = (i % DCH) * 16
            p = pos_buf[r, pl.ds(off, 16)]
            for b in range(BATCH):
                x_buf[r, b, pl.ds(off, 16)] = x_buf[r, b, pl.ds(off, 16)] + p
            return 0

        lax.fori_loop(0, CHUNK * DCH, add_body, 0)
        pltpu.sync_copy(x_buf, out_hbm.at[pl.ds(base, CHUNK)])
        return 0

    lax.fori_loop(0, NCHUNK, chunk_body, 0)


def _sc_kernel(x, pos_embedding):
    mesh = plsc.VectorSubcoreMesh(core_axis_name="c", subcore_axis_name="s")
    f = functools.partial(
        pl.kernel,
        out_type=jax.ShapeDtypeStruct((SEQ, BATCH, D), jnp.float32),
        mesh=mesh,
        scratch_types=[
            pltpu.VMEM((CHUNK, BATCH, D), jnp.float32),
            pltpu.VMEM((CHUNK, D), jnp.float32),
        ],
    )(_sc_body)
    return f(x, pos_embedding)


def kernel(x, pos_embedding):
    return _sc_kernel(x, pos_embedding)


# speedup vs baseline: 1.5344x; 1.5344x over previous
import functools

import jax
import jax.numpy as jnp
from jax import lax
from jax.experimental import pallas as pl
from jax.experimental.pallas import tpu as pltpu
from jax.experimental.pallas import tpu_sc as plsc

SEQ = 8192
BATCH = 4
D = 1024
NW = 32
ROWS_PER_W = SEQ // NW
CHUNK = 8
NCHUNK = ROWS_PER_W // CHUNK
DCH = D // 16


def _sc_body(x_hbm, pos_hbm, out_hbm,
             xb0, pb0, xb1, pb1, sx0, sp0, so0, sx1, sp1, so1):
    wid = lax.axis_index("s") * 2 + lax.axis_index("c")
    row0 = wid * ROWS_PER_W
    xb, pb = [xb0, xb1], [pb0, pb1]
    sx, sp, so = [sx0, sx1], [sp0, sp1], [so0, so1]

    def in_copies(ci, k):
        base = row0 + ci * CHUNK
        return (pltpu.make_async_copy(x_hbm.at[pl.ds(base, CHUNK)], xb[k], sx[k]),
                pltpu.make_async_copy(pos_hbm.at[pl.ds(base, CHUNK)], pb[k], sp[k]))

    def out_copy(ci, k):
        base = row0 + ci * CHUNK
        return pltpu.make_async_copy(xb[k], out_hbm.at[pl.ds(base, CHUNK)], so[k])

    def compute(k):
        def add_body(i, _):
            r = i // DCH
            off = (i % DCH) * 16
            p = pb[k][r, pl.ds(off, 16)]
            for b in range(BATCH):
                xb[k][r, b, pl.ds(off, 16)] = xb[k][r, b, pl.ds(off, 16)] + p
            return 0

        lax.fori_loop(0, CHUNK * DCH, add_body, 0)

    cx, cp = in_copies(0, 0)
    cx.start(); cp.start()
    for ci in range(NCHUNK):
        k = ci % 2
        o = (ci + 1) % 2
        if ci + 1 < NCHUNK:
            if ci >= 1:
                out_copy(ci - 1, o).wait()
            nx, np_ = in_copies(ci + 1, o)
            nx.start(); np_.start()
        cxk, cpk = in_copies(ci, k)
        cxk.wait(); cpk.wait()
        compute(k)
        out_copy(ci, k).start()
    out_copy(NCHUNK - 2, 0 if (NCHUNK - 2) % 2 == 0 else 1).wait()
    out_copy(NCHUNK - 1, (NCHUNK - 1) % 2).wait()


def _sc_kernel(x, pos_embedding):
    mesh = plsc.VectorSubcoreMesh(core_axis_name="c", subcore_axis_name="s")
    f = functools.partial(
        pl.kernel,
        out_type=jax.ShapeDtypeStruct((SEQ, BATCH, D), jnp.float32),
        mesh=mesh,
        scratch_types=[
            pltpu.VMEM((CHUNK, BATCH, D), jnp.float32),
            pltpu.VMEM((CHUNK, D), jnp.float32),
            pltpu.VMEM((CHUNK, BATCH, D), jnp.float32),
            pltpu.VMEM((CHUNK, D), jnp.float32),
            pltpu.SemaphoreType.DMA,
            pltpu.SemaphoreType.DMA,
            pltpu.SemaphoreType.DMA,
            pltpu.SemaphoreType.DMA,
            pltpu.SemaphoreType.DMA,
            pltpu.SemaphoreType.DMA,
        ],
    )(_sc_body)
    return f(x, pos_embedding)


def kernel(x, pos_embedding):
    return _sc_kernel(x, pos_embedding)


# speedup vs baseline: 1.9605x; 1.2777x over previous
import functools

import jax
import jax.numpy as jnp
from jax import lax
from jax.experimental import pallas as pl
from jax.experimental.pallas import tpu as pltpu
from jax.experimental.pallas import tpu_sc as plsc

SEQ = 8192
BATCH = 4
D = 1024
NW = 32
K_SC = 2560
ROWS_PER_W = K_SC // NW
CHUNK = 8
NCHUNK = ROWS_PER_W // CHUNK
DCH = D // 16
BS_TC = 512


def _sc_body(x_hbm, pos_hbm, out_hbm,
             xb0, pb0, xb1, pb1, sx0, sp0, so0, sx1, sp1, so1):
    wid = lax.axis_index("s") * 2 + lax.axis_index("c")
    row0 = wid * ROWS_PER_W
    xb, pb = [xb0, xb1], [pb0, pb1]
    sx, sp, so = [sx0, sx1], [sp0, sp1], [so0, so1]

    def in_copies(ci, k):
        base = row0 + ci * CHUNK
        return (pltpu.make_async_copy(x_hbm.at[pl.ds(base, CHUNK)], xb[k], sx[k]),
                pltpu.make_async_copy(pos_hbm.at[pl.ds(base, CHUNK)], pb[k], sp[k]))

    def out_copy(ci, k):
        base = row0 + ci * CHUNK
        return pltpu.make_async_copy(xb[k], out_hbm.at[pl.ds(base, CHUNK)], so[k])

    def compute(k):
        def add_body(i, _):
            r = i // DCH
            off = (i % DCH) * 16
            p = pb[k][r, pl.ds(off, 16)]
            for b in range(BATCH):
                xb[k][r, b, pl.ds(off, 16)] = xb[k][r, b, pl.ds(off, 16)] + p
            return 0

        lax.fori_loop(0, CHUNK * DCH, add_body, 0)

    cx, cp = in_copies(0, 0)
    cx.start(); cp.start()
    for ci in range(NCHUNK):
        k = ci % 2
        o = (ci + 1) % 2
        if ci + 1 < NCHUNK:
            if ci >= 1:
                out_copy(ci - 1, o).wait()
            nx, np_ = in_copies(ci + 1, o)
            nx.start(); np_.start()
        cxk, cpk = in_copies(ci, k)
        cxk.wait(); cpk.wait()
        compute(k)
        out_copy(ci, k).start()
    out_copy(NCHUNK - 2, (NCHUNK - 2) % 2).wait()
    out_copy(NCHUNK - 1, (NCHUNK - 1) % 2).wait()


def _sc_part(x, pos_embedding):
    mesh = plsc.VectorSubcoreMesh(core_axis_name="c", subcore_axis_name="s")
    f = functools.partial(
        pl.kernel,
        out_type=jax.ShapeDtypeStruct((K_SC, BATCH, D), jnp.float32),
        mesh=mesh,
        scratch_types=[
            pltpu.VMEM((CHUNK, BATCH, D), jnp.float32),
            pltpu.VMEM((CHUNK, D), jnp.float32),
            pltpu.VMEM((CHUNK, BATCH, D), jnp.float32),
            pltpu.VMEM((CHUNK, D), jnp.float32),
            pltpu.SemaphoreType.DMA,
            pltpu.SemaphoreType.DMA,
            pltpu.SemaphoreType.DMA,
            pltpu.SemaphoreType.DMA,
            pltpu.SemaphoreType.DMA,
            pltpu.SemaphoreType.DMA,
        ],
    )(_sc_body)
    return f(x, pos_embedding)


def _tc_add_body(x_ref, pos_ref, out_ref):
    out_ref[...] = x_ref[...] + pos_ref[...][:, None, :]


def _tc_part(x, pos_embedding):
    off = K_SC // BS_TC
    grid = ((SEQ - K_SC) // BS_TC,)
    return pl.pallas_call(
        _tc_add_body,
        grid=grid,
        in_specs=[
            pl.BlockSpec((BS_TC, BATCH, D), lambda i: (i + off, 0, 0)),
            pl.BlockSpec((BS_TC, D), lambda i: (i + off, 0)),
        ],
        out_specs=pl.BlockSpec((BS_TC, BATCH, D), lambda i: (i + off, 0, 0)),
        out_shape=jax.ShapeDtypeStruct((SEQ, BATCH, D), jnp.float32),
    )(x, pos_embedding)


def kernel(x, pos_embedding):
    sc = _sc_part(x, pos_embedding)
    tc = _tc_part(x, pos_embedding)
    return lax.dynamic_update_slice(tc, sc, (0, 0, 0))


# speedup vs baseline: 2.8863x; 1.4722x over previous
import jax
import jax.numpy as jnp
from jax.experimental import pallas as pl


_BS = 512


def _add_body(x_ref, pos_ref, out_ref):
    out_ref[...] = x_ref[...] + pos_ref[...][:, None, :]


def kernel(x, pos_embedding):
    seq, batch, d = x.shape
    grid = (seq // _BS,)
    return pl.pallas_call(
        _add_body,
        grid=grid,
        in_specs=[
            pl.BlockSpec((_BS, batch, d), lambda i: (i, 0, 0)),
            pl.BlockSpec((_BS, d), lambda i: (i, 0)),
        ],
        out_specs=pl.BlockSpec((_BS, batch, d), lambda i: (i, 0, 0)),
        out_shape=jax.ShapeDtypeStruct((seq, batch, d), x.dtype),
    )(x, pos_embedding[:seq])
